# baseline (speedup 1.0000x reference)
import functools

import jax
import jax.numpy as jnp
from jax.experimental import pallas as pl

_BLOCK_N = 2048


def _assign_kernel(v_ref, c_ref, out_ref):
    v = v_ref[:]
    norm = jnp.sqrt(jnp.sum(v * v, axis=1, keepdims=True))
    vn = v / jnp.maximum(norm, 1e-12)
    sim = jax.lax.dot_general(
        vn, c_ref[:], (((1,), (1,)), ((), ())),
        preferred_element_type=jnp.float32)
    out_ref[:] = jnp.argmax(sim, axis=1).astype(jnp.int32)


@functools.partial(jax.jit, static_argnames=())
def kernel(vectors, centroids):
    n, d = vectors.shape
    k = centroids.shape[0]
    grid = (n // _BLOCK_N,)
    return pl.pallas_call(
        _assign_kernel,
        grid=grid,
        in_specs=[
            pl.BlockSpec((_BLOCK_N, d), lambda i: (i, 0)),
            pl.BlockSpec((k, d), lambda i: (0, 0)),
        ],
        out_specs=pl.BlockSpec((_BLOCK_N,), lambda i: (i,)),
        out_shape=jax.ShapeDtypeStruct((n,), jnp.int32),
    )(vectors, centroids)
